# baseline (speedup 1.0000x reference)
import jax
import jax.numpy as jnp
from jax.experimental import pallas as pl
from jax.experimental.pallas import tpu as pltpu

_EPS = 1e-5
_HWP = 640


def _pool_kernel(x_ref, o_ref):
    BT, CT, _ = x_ref.shape
    pooled = []
    for bi in range(BT):
        T = jnp.transpose(x_ref[bi], (1, 0))
        W5 = T[:576].reshape(3, 8, 3, 8, CT)
        m1 = jnp.max(W5, axis=1)
        m2 = jnp.max(m1, axis=2)
        pooled.append(m2)
    o_ref[...] = jnp.stack(pooled, axis=2)


def _bn_linear_kernel(p_ref, w_ref, b_ref, o_ref):
    PP, C, _ = w_ref.shape
    _, _, B, _ = p_ref.shape
    x = p_ref[...].reshape(PP, B, C)

    inv_cnt = 1.0 / float(PP * B)
    mean = jnp.sum(x, axis=(0, 1), keepdims=True) * inv_cnt
    diff = x - mean
    var = jnp.sum(diff * diff, axis=(0, 1), keepdims=True) * inv_cnt
    nrm = diff * jax.lax.rsqrt(var + _EPS)

    acc = b_ref[...]
    for p in range(PP):
        acc = acc + jnp.dot(nrm[p], w_ref[p], preferred_element_type=jnp.float32)
    o_ref[...] = acc


def kernel(feat, w, b):
    B, C, H, W = feat.shape
    P = 3
    PP = P * P
    HW = H * W
    N = w.shape[1]

    x2 = jnp.pad(feat.reshape(B, C, HW), ((0, 0), (0, 0), (0, _HWP - HW)))

    NUM_CORES = 2
    BT = 8
    STEPS = B // (NUM_CORES * BT)
    CT = C // 2

    pooled = pl.pallas_call(
        _pool_kernel,
        out_shape=jax.ShapeDtypeStruct((P, P, B, C), jnp.float32),
        grid=(NUM_CORES, STEPS, 2),
        in_specs=[
            pl.BlockSpec((BT, CT, _HWP), lambda k, i, c: (k * STEPS + i, c, 0)),
        ],
        out_specs=pl.BlockSpec((P, P, BT, CT),
                               lambda k, i, c: (0, 0, k * STEPS + i, c)),
        compiler_params=pltpu.CompilerParams(
            dimension_semantics=("parallel", "arbitrary", "arbitrary"),
        ),
    )(x2)

    n_pad = ((N + 127) // 128) * 128
    w_r = (jnp.pad(w, ((0, 0), (0, n_pad - N)))
              .reshape(C, PP, n_pad)
              .transpose(1, 0, 2))
    b_pad = jnp.pad(b, (0, n_pad - N)).reshape(1, n_pad)

    out = pl.pallas_call(
        _bn_linear_kernel,
        out_shape=jax.ShapeDtypeStruct((B, n_pad), jnp.float32),
        grid=(1,),
        in_specs=[
            pl.BlockSpec((P, P, B, C), lambda k: (0, 0, 0, 0)),
            pl.BlockSpec((PP, C, n_pad), lambda k: (0, 0, 0)),
            pl.BlockSpec((1, n_pad), lambda k: (0, 0)),
        ],
        out_specs=pl.BlockSpec((B, n_pad), lambda k: (0, 0)),
        compiler_params=pltpu.CompilerParams(
            dimension_semantics=("arbitrary",),
        ),
    )(pooled, w_r, b_pad)

    return out[:, :N]


# speedup vs baseline: 1.5491x; 1.5491x over previous
import jax
import jax.numpy as jnp
from jax.experimental import pallas as pl
from jax.experimental.pallas import tpu as pltpu

_EPS = 1e-5


def _pool_half(x_ref, bi):
    T = jnp.transpose(x_ref[bi], (1, 0))
    W5 = T.reshape(3, 8, 3, 8, x_ref.shape[1])
    m1 = jnp.max(W5, axis=1)
    return jnp.max(m1, axis=2)


def _pool_kernel(xa_ref, xb_ref, o_ref):
    BT = xa_ref.shape[0]
    pooled = []
    for bi in range(BT):
        pooled.append(jnp.concatenate(
            [_pool_half(xa_ref, bi), _pool_half(xb_ref, bi)], axis=-1))
    o_ref[...] = jnp.stack(pooled, axis=2)


def _bn_linear_kernel(p_ref, w_ref, b_ref, o_ref):
    PP, C, _ = w_ref.shape
    _, _, B, _ = p_ref.shape
    x = p_ref[...].reshape(PP, B, C)

    inv_cnt = 1.0 / float(PP * B)
    mean = jnp.sum(x, axis=(0, 1), keepdims=True) * inv_cnt
    diff = x - mean
    var = jnp.sum(diff * diff, axis=(0, 1), keepdims=True) * inv_cnt
    nrm = diff * jax.lax.rsqrt(var + _EPS)

    acc = b_ref[...]
    for p in range(PP):
        acc = acc + jnp.dot(nrm[p], w_ref[p], preferred_element_type=jnp.float32)
    o_ref[...] = acc


def kernel(feat, w, b):
    B, C, H, W = feat.shape
    P = 3
    PP = P * P
    HW = H * W
    N = w.shape[1]

    x2 = feat.reshape(B, C, HW)

    NUM_CORES = 2
    BT = 8
    STEPS = B // (NUM_CORES * BT)
    CH = C // 2

    pooled = pl.pallas_call(
        _pool_kernel,
        out_shape=jax.ShapeDtypeStruct((P, P, B, C), jnp.float32),
        grid=(NUM_CORES, STEPS),
        in_specs=[
            pl.BlockSpec((BT, CH, HW), lambda k, i: (k * STEPS + i, 0, 0)),
            pl.BlockSpec((BT, CH, HW), lambda k, i: (k * STEPS + i, 1, 0)),
        ],
        out_specs=pl.BlockSpec((P, P, BT, C),
                               lambda k, i: (0, 0, k * STEPS + i, 0)),
        compiler_params=pltpu.CompilerParams(
            dimension_semantics=("parallel", "arbitrary"),
        ),
    )(x2, x2)

    n_pad = ((N + 127) // 128) * 128
    w_r = (jnp.pad(w, ((0, 0), (0, n_pad - N)))
              .reshape(C, PP, n_pad)
              .transpose(1, 0, 2))
    b_pad = jnp.pad(b, (0, n_pad - N)).reshape(1, n_pad)

    out = pl.pallas_call(
        _bn_linear_kernel,
        out_shape=jax.ShapeDtypeStruct((B, n_pad), jnp.float32),
        grid=(1,),
        in_specs=[
            pl.BlockSpec((P, P, B, C), lambda k: (0, 0, 0, 0)),
            pl.BlockSpec((PP, C, n_pad), lambda k: (0, 0, 0)),
            pl.BlockSpec((1, n_pad), lambda k: (0, 0)),
        ],
        out_specs=pl.BlockSpec((B, n_pad), lambda k: (0, 0)),
        compiler_params=pltpu.CompilerParams(
            dimension_semantics=("arbitrary",),
        ),
    )(pooled, w_r, b_pad)

    return out[:, :N]


# speedup vs baseline: 1.5639x; 1.0096x over previous
import jax
import jax.numpy as jnp
from jax.experimental import pallas as pl
from jax.experimental.pallas import tpu as pltpu

_EPS = 1e-5


def _pool_half(x_ref, bi):
    T = jnp.transpose(x_ref[bi], (1, 0))
    W5 = T.reshape(3, 8, 3, 8, x_ref.shape[1])
    m1 = jnp.max(W5, axis=1)
    return jnp.max(m1, axis=2)


def _pool_kernel(x_ref, o_ref):
    BT = x_ref.shape[0]
    pooled = [_pool_half(x_ref, bi) for bi in range(BT)]
    o_ref[...] = jnp.stack(pooled, axis=2)


def _bn_linear_kernel(p_ref, w_ref, b_ref, o_ref):
    PP, C, _ = w_ref.shape
    _, _, B, _ = p_ref.shape
    x = p_ref[...].reshape(PP, B, C)

    inv_cnt = 1.0 / float(PP * B)
    mean = jnp.sum(x, axis=(0, 1), keepdims=True) * inv_cnt
    diff = x - mean
    var = jnp.sum(diff * diff, axis=(0, 1), keepdims=True) * inv_cnt
    nrm = diff * jax.lax.rsqrt(var + _EPS)

    acc = b_ref[...]
    for p in range(PP):
        acc = acc + jnp.dot(nrm[p], w_ref[p], preferred_element_type=jnp.float32)
    o_ref[...] = acc


def kernel(feat, w, b):
    B, C, H, W = feat.shape
    P = 3
    PP = P * P
    HW = H * W
    N = w.shape[1]

    x2 = feat.reshape(B, C, HW)

    NUM_CORES = 2
    BT = B // NUM_CORES
    STEPS = 1

    pooled = pl.pallas_call(
        _pool_kernel,
        out_shape=jax.ShapeDtypeStruct((P, P, B, C), jnp.float32),
        grid=(NUM_CORES, STEPS),
        in_specs=[
            pl.BlockSpec((BT, C, HW), lambda k, i: (k * STEPS + i, 0, 0)),
        ],
        out_specs=pl.BlockSpec((P, P, BT, C),
                               lambda k, i: (0, 0, k * STEPS + i, 0)),
        compiler_params=pltpu.CompilerParams(
            dimension_semantics=("parallel", "arbitrary"),
        ),
    )(x2)

    w_r = jnp.transpose(w.reshape(C, PP, N), (1, 0, 2))
    b2 = b.reshape(1, N)

    out = pl.pallas_call(
        _bn_linear_kernel,
        out_shape=jax.ShapeDtypeStruct((B, N), jnp.float32),
        grid=(1,),
        in_specs=[
            pl.BlockSpec((P, P, B, C), lambda k: (0, 0, 0, 0)),
            pl.BlockSpec((PP, C, N), lambda k: (0, 0, 0)),
            pl.BlockSpec((1, N), lambda k: (0, 0)),
        ],
        out_specs=pl.BlockSpec((B, N), lambda k: (0, 0)),
        compiler_params=pltpu.CompilerParams(
            dimension_semantics=("arbitrary",),
        ),
    )(pooled, w_r, b2)

    return out
